# baseline (speedup 1.0000x reference)
import functools

import jax
import jax.numpy as jnp
from jax import lax
from jax.experimental import pallas as pl
from jax.experimental.pallas import tpu as pltpu
from jax.experimental.pallas import tpu_sc as plsc

_NC = 2
_NS = 16
_NW = _NC * _NS
_LANES = 16
_NSLOT = 3


def kernel(x, pe):
    B, S, D = x.shape
    R = 8
    SW = S // _NW
    n_sub = SW // R
    n_vec = R * D // _LANES
    n_col = D // _LANES

    mesh = plsc.VectorSubcoreMesh(core_axis_name="c", subcore_axis_name="s")

    scratch = (
        [pltpu.VMEM((R, D), jnp.float32) for _ in range(_NSLOT * B)]
        + [pltpu.VMEM((R, D), jnp.float32) for _ in range(2)]
        + [pltpu.SemaphoreType.DMA for _ in range(_NSLOT)]
        + [pltpu.SemaphoreType.DMA for _ in range(_NSLOT)]
        + [pltpu.SemaphoreType.DMA for _ in range(2)]
        + [pltpu.SemaphoreType.DMA]
        + [pltpu.VMEM_SHARED((_NS * 16, 1024), jnp.float32)]
    )

    @functools.partial(
        pl.kernel,
        out_type=jax.ShapeDtypeStruct((B, S, D), jnp.float32),
        mesh=mesh,
        scratch_types=scratch,
    )
    def run(x_hbm, pe_hbm, out_hbm, *bufs):
        xb = [list(bufs[s * B:(s + 1) * B]) for s in range(_NSLOT)]
        o = _NSLOT * B
        peb = list(bufs[o:o + 2])
        o += 2
        xsem = list(bufs[o:o + _NSLOT])
        osem = list(bufs[o + _NSLOT:o + 2 * _NSLOT])
        psem = list(bufs[o + 2 * _NSLOT:o + 2 * _NSLOT + 2])
        ssem = bufs[o + 2 * _NSLOT + 2]
        smem_shared = bufs[o + 2 * _NSLOT + 3]

        wid = lax.axis_index("s") * _NC + lax.axis_index("c")
        base = wid * SW

        def pe_block(t):
            return pe_hbm.at[pl.ds(base + t * R, R)]

        def x_block(t, b):
            return x_hbm.at[b].at[pl.ds(base + t * R, R)]

        def out_block(t, b):
            return out_hbm.at[b].at[pl.ds(base + t * R, R)]

        pe_cp = [None] * n_sub
        x_cp = [[None] * B for _ in range(n_sub)]
        o_cp = [[None] * B for _ in range(n_sub)]
        pe_cp[0] = pltpu.async_copy(pe_block(0), peb[0], psem[0])
        for b in range(B):
            x_cp[0][b] = pltpu.async_copy(x_block(0, b), xb[0][b], xsem[0])

        for t in range(n_sub):
            slot = t % _NSLOT
            nt = t + 1
            if nt < n_sub:
                ns = nt % _NSLOT
                pe_cp[nt] = pltpu.async_copy(
                    pe_block(nt), peb[nt % 2], psem[nt % 2])
                for b in range(B):
                    if t - 2 >= 0 and b >= 2:
                        o_cp[t - 2][b].wait()
                    x_cp[nt][b] = pltpu.async_copy(
                        x_block(nt, b), xb[ns][b], xsem[ns])
            pe_cp[t].wait()
            for b in range(B):
                x_cp[t][b].wait()

            xs = xb[slot]
            pr = peb[t % 2]

            @plsc.parallel_loop(0, n_vec, step=1, unroll=4)
            def add_body(i):
                r = i // n_col
                c = (i % n_col) * _LANES
                sl = pl.ds(c, _LANES)
                v = pr[r, sl]
                for xrb in xs:
                    plsc.addupdate(xrb.at[r, sl], v)

            sid = lax.axis_index("s")
            s_cp = {}
            for b in range(2):
                if t - 1 >= 0:
                    o_cp[t - 1][b].wait()
                off = sid * 16 + b * R
                s_cp[b] = pltpu.async_copy(
                    xs[b], smem_shared.at[pl.ds(off, R)], ssem)
            for b in range(2):
                s_cp[b].wait()
                off = sid * 16 + b * R
                o_cp[t][b] = pltpu.async_copy(
                    smem_shared.at[pl.ds(off, R)], out_block(t, b), osem[slot])
            for b in range(2, B):
                o_cp[t][b] = pltpu.async_copy(
                    xs[b], out_block(t, b), osem[slot])

        for b in range(2):
            o_cp[n_sub - 1][b].wait()
        for t in (n_sub - 3, n_sub - 2, n_sub - 1):
            if t >= 0:
                for b in range(2, B):
                    o_cp[t][b].wait()

    return run(x, pe)


# speedup vs baseline: 1.0102x; 1.0102x over previous
import functools

import jax
import jax.numpy as jnp
from jax import lax
from jax.experimental import pallas as pl
from jax.experimental.pallas import tpu as pltpu
from jax.experimental.pallas import tpu_sc as plsc

_NC = 2
_NS = 16
_NW = _NC * _NS
_LANES = 16
_NSLOT = 3


def kernel(x, pe):
    B, S, D = x.shape
    R = 8
    SW = S // _NW
    n_sub = SW // R
    n_vec = R * D // _LANES
    n_col = D // _LANES

    mesh = plsc.VectorSubcoreMesh(core_axis_name="c", subcore_axis_name="s")

    scratch = (
        [pltpu.VMEM((R, D), jnp.float32) for _ in range(_NSLOT * B)]
        + [pltpu.VMEM((R, D), jnp.float32) for _ in range(2)]
        + [pltpu.SemaphoreType.DMA for _ in range(_NSLOT)]
        + [pltpu.SemaphoreType.DMA for _ in range(_NSLOT)]
        + [pltpu.SemaphoreType.DMA for _ in range(2)]
    )

    @functools.partial(
        pl.kernel,
        out_type=jax.ShapeDtypeStruct((B, S, D), jnp.float32),
        mesh=mesh,
        scratch_types=scratch,
    )
    def run(x_hbm, pe_hbm, out_hbm, *bufs):
        xb = [list(bufs[s * B:(s + 1) * B]) for s in range(_NSLOT)]
        o = _NSLOT * B
        peb = list(bufs[o:o + 2])
        o += 2
        xsem = list(bufs[o:o + _NSLOT])
        osem = list(bufs[o + _NSLOT:o + 2 * _NSLOT])
        psem = list(bufs[o + 2 * _NSLOT:o + 2 * _NSLOT + 2])

        wid = lax.axis_index("s") * _NC + lax.axis_index("c")
        base = wid * SW

        def pe_block(t):
            return pe_hbm.at[pl.ds(base + t * R, R)]

        def x_block(t, b):
            return x_hbm.at[b].at[pl.ds(base + t * R, R)]

        def out_block(t, b):
            return out_hbm.at[b].at[pl.ds(base + t * R, R)]

        pe_cp = [None] * n_sub
        x_cp = [[None] * B for _ in range(n_sub)]
        o_cp = [[None] * B for _ in range(n_sub)]
        pe_cp[0] = pltpu.async_copy(pe_block(0), peb[0], psem[0])
        for b in range(B):
            x_cp[0][b] = pltpu.async_copy(x_block(0, b), xb[0][b], xsem[0])

        for t in range(n_sub):
            slot = t % _NSLOT
            nt = t + 1
            if nt < n_sub:
                ns = nt % _NSLOT
                pe_cp[nt] = pltpu.async_copy(
                    pe_block(nt), peb[nt % 2], psem[nt % 2])
                for b in range(B):
                    if t - 2 >= 0:
                        o_cp[t - 2][b].wait()
                    x_cp[nt][b] = pltpu.async_copy(
                        x_block(nt, b), xb[ns][b], xsem[ns])
            pe_cp[t].wait()
            for b in range(B):
                x_cp[t][b].wait()

            xs = xb[slot]
            pr = peb[t % 2]

            @plsc.parallel_loop(0, n_vec, step=1, unroll=4)
            def add_body(i):
                r = i // n_col
                c = (i % n_col) * _LANES
                sl = pl.ds(c, _LANES)
                v = pr[r, sl]
                for xrb in xs:
                    plsc.addupdate(xrb.at[r, sl], v)

            for b in range(B):
                o_cp[t][b] = pltpu.async_copy(
                    xs[b], out_block(t, b), osem[slot])

        for t in (n_sub - 3, n_sub - 2, n_sub - 1):
            if t >= 0:
                for b in range(B):
                    o_cp[t][b].wait()

    return run(x, pe)
